# baseline (speedup 1.0000x reference)
import jax
import jax.numpy as jnp
from jax.experimental import pallas as pl
from jax.experimental.pallas import tpu as pltpu

_CAPACITY = 65536
_EMBED = 256
_PTR = 0
_NFEAT = 4096

_NCH = 4
_CROWS = _CAPACITY // _NCH
_NBUF = 3
_SLACK = 2


def _body(f_hbm, m_hbm, o_hbm, fvmem, bufs, rowbuf,
          in_sems, out_sems, f_sem, row_sem):
    def in_copy(i):
        return pltpu.make_async_copy(
            m_hbm.at[pl.ds(i * _CROWS, _CROWS), :],
            bufs.at[i % _NBUF],
            in_sems.at[i % _NBUF],
        )

    def out_copy(i):
        return pltpu.make_async_copy(
            bufs.at[i % _NBUF],
            o_hbm.at[pl.ds(i * _CROWS, _CROWS), :],
            out_sems.at[i % _NBUF],
        )

    fcopy = pltpu.make_async_copy(f_hbm, fvmem, f_sem)
    fcopy.start()
    for b in range(_NBUF):
        in_copy(b).start()
    fcopy.wait()
    rowbuf[...] = jnp.sum(fvmem[...], axis=0, keepdims=True) * (1.0 / _NFEAT)

    patch = pltpu.make_async_copy(rowbuf, o_hbm.at[pl.ds(_PTR, 1), :], row_sem)
    for i in range(_NCH):
        in_copy(i).wait()
        out_copy(i).start()
        j = i - _SLACK
        if j >= 0:
            out_copy(j).wait()
            if j + _NBUF < _NCH:
                in_copy(j + _NBUF).start()
            if j == _PTR // _CROWS:
                patch.start()
    for j in range(max(0, _NCH - _SLACK), _NCH):
        out_copy(j).wait()
    patch.wait()


def kernel(feature, memory):
    return pl.pallas_call(
        _body,
        in_specs=[
            pl.BlockSpec(memory_space=pl.ANY),
            pl.BlockSpec(memory_space=pl.ANY),
        ],
        out_specs=pl.BlockSpec(memory_space=pl.ANY),
        out_shape=jax.ShapeDtypeStruct((_CAPACITY, _EMBED), jnp.float32),
        scratch_shapes=[
            pltpu.VMEM((_NFEAT, _EMBED), jnp.float32),
            pltpu.VMEM((_NBUF, _CROWS, _EMBED), jnp.float32),
            pltpu.VMEM((1, _EMBED), jnp.float32),
            pltpu.SemaphoreType.DMA((_NBUF,)),
            pltpu.SemaphoreType.DMA((_NBUF,)),
            pltpu.SemaphoreType.DMA,
            pltpu.SemaphoreType.DMA,
        ],
    )(feature, memory)


# speedup vs baseline: 1.0012x; 1.0012x over previous
import jax
import jax.numpy as jnp
from jax.experimental import pallas as pl
from jax.experimental.pallas import tpu as pltpu

_CAPACITY = 65536
_EMBED = 256
_PTR = 0
_NFEAT = 4096

_NCH = 4
_CROWS = _CAPACITY // _NCH
_NBUF = 3
_SLACK = 1


def _body(f_hbm, m_hbm, o_hbm, fvmem, bufs, rowbuf,
          in_sems, out_sems, f_sem, row_sem):
    def in_copy(i):
        return pltpu.make_async_copy(
            m_hbm.at[pl.ds(i * _CROWS, _CROWS), :],
            bufs.at[i % _NBUF],
            in_sems.at[i % _NBUF],
        )

    def out_copy(i):
        return pltpu.make_async_copy(
            bufs.at[i % _NBUF],
            o_hbm.at[pl.ds(i * _CROWS, _CROWS), :],
            out_sems.at[i % _NBUF],
        )

    fcopy = pltpu.make_async_copy(f_hbm, fvmem, f_sem)
    fcopy.start()
    for b in range(_NBUF):
        in_copy(b).start()
    fcopy.wait()
    rowbuf[...] = jnp.sum(fvmem[...], axis=0, keepdims=True) * (1.0 / _NFEAT)

    patch = pltpu.make_async_copy(rowbuf, o_hbm.at[pl.ds(_PTR, 1), :], row_sem)
    for i in range(_NCH):
        in_copy(i).wait()
        out_copy(i).start()
        j = i - _SLACK
        if j >= 0:
            out_copy(j).wait()
            if j + _NBUF < _NCH:
                in_copy(j + _NBUF).start()
            if j == _PTR // _CROWS:
                patch.start()
    for j in range(max(0, _NCH - _SLACK), _NCH):
        out_copy(j).wait()
    patch.wait()


def kernel(feature, memory):
    return pl.pallas_call(
        _body,
        in_specs=[
            pl.BlockSpec(memory_space=pl.ANY),
            pl.BlockSpec(memory_space=pl.ANY),
        ],
        out_specs=pl.BlockSpec(memory_space=pl.ANY),
        out_shape=jax.ShapeDtypeStruct((_CAPACITY, _EMBED), jnp.float32),
        scratch_shapes=[
            pltpu.VMEM((_NFEAT, _EMBED), jnp.float32),
            pltpu.VMEM((_NBUF, _CROWS, _EMBED), jnp.float32),
            pltpu.VMEM((1, _EMBED), jnp.float32),
            pltpu.SemaphoreType.DMA((_NBUF,)),
            pltpu.SemaphoreType.DMA((_NBUF,)),
            pltpu.SemaphoreType.DMA,
            pltpu.SemaphoreType.DMA,
        ],
    )(feature, memory)
